# baseline (speedup 1.0000x reference)
import functools

import jax
import jax.numpy as jnp
from jax import lax
from jax.experimental import pallas as pl
from jax.experimental.pallas import tpu as pltpu
from jax.experimental.pallas import tpu_sc as plsc

N = 10000
E = 320000
D = 128
NC = 2
NS = 16
NW = NC * NS
EPW = E // NW
CHUNK = 80
NITER = EPW // CHUNK
NPAD = 10240
RPW = NPAD // NS
ZR = 32
DPW = NPAD // NS
ACHUNK = 40
ANITER = EPW // ACHUNK
NB = 5
SB = 2
UNROLL = 10
NGRP = ANITER // UNROLL

_mesh = plsc.VectorSubcoreMesh(core_axis_name="c", subcore_axis_name="s")


@functools.partial(
    pl.kernel,
    out_type=jax.ShapeDtypeStruct((NC, NPAD), jnp.float32),
    mesh=_mesh,
    scratch_types=[
        pltpu.VMEM((EPW,), jnp.int32),
        pltpu.VMEM((EPW,), jnp.float32),
        pltpu.VMEM((NPAD,), jnp.float32),
        pltpu.VMEM_SHARED((NS, NPAD), jnp.float32),
        pltpu.VMEM((DPW,), jnp.float32),
        pltpu.VMEM((DPW,), jnp.float32),
        pltpu.SemaphoreType.DMA,
        pltpu.SemaphoreType.DMA,
    ],
    compiler_params=pltpu.CompilerParams(needs_layout_passes=False),
)
def _deg_kernel(dst_hbm, ew_hbm, pdeg_hbm, dstb, ewb, ldeg, share, rowb, accb,
                dsem, esem):
    c = lax.axis_index("c")
    s = lax.axis_index("s")
    wid = c * NS + s
    zero16 = jnp.zeros((16,), jnp.float32)

    def zb(i, _):
        ldeg[pl.ds(i * 16, 16)] = zero16
        return 0
    lax.fori_loop(0, NPAD // 16, zb, 0)

    ebase = wid * EPW
    pltpu.async_copy(dst_hbm.at[pl.ds(ebase, EPW)], dstb, dsem)
    pltpu.async_copy(ew_hbm.at[pl.ds(ebase, EPW)], ewb, esem)
    pltpu.make_async_copy(dst_hbm.at[pl.ds(ebase, EPW)], dstb, dsem).wait()
    pltpu.make_async_copy(ew_hbm.at[pl.ds(ebase, EPW)], ewb, esem).wait()

    @plsc.parallel_loop(0, EPW // 16, step=1, unroll=4)
    def body(i):
        d16 = dstb[pl.ds(i * 16, 16)]
        w16 = ewb[pl.ds(i * 16, 16)]
        plsc.addupdate_scatter(ldeg, [d16], w16)

    pltpu.sync_copy(ldeg, share.at[s])
    plsc.subcore_barrier()
    base = s * DPW

    def za(k, _):
        accb[pl.ds(k * 16, 16)] = zero16
        return 0
    lax.fori_loop(0, DPW // 16, za, 0)

    def rsum(r, _):
        pltpu.sync_copy(share.at[r, pl.ds(base, DPW)], rowb)

        def vadd(k, _):
            accb[pl.ds(k * 16, 16)] = accb[pl.ds(k * 16, 16)] + rowb[pl.ds(k * 16, 16)]
            return 0
        lax.fori_loop(0, DPW // 16, vadd, 0)
        return 0
    lax.fori_loop(0, NS, rsum, 0)

    pltpu.sync_copy(accb, pdeg_hbm.at[c, pl.ds(base, DPW)])


@functools.partial(
    pl.kernel,
    out_type=jax.ShapeDtypeStruct((NC, NPAD, D), jnp.float32),
    mesh=_mesh,
    scratch_types=[
        pltpu.VMEM_SHARED((NPAD, D), jnp.float32),
        pltpu.VMEM((NB, ACHUNK), jnp.int32),
        pltpu.VMEM((NB, 2, ACHUNK), jnp.int32),
        pltpu.VMEM((NB, ACHUNK, D), jnp.float32),
        pltpu.VMEM((SB, ACHUNK, D), jnp.float32),
        pltpu.SemaphoreType.DMA((NB,)),
        pltpu.SemaphoreType.DMA((SB,)),
        pltpu.SemaphoreType.DMA((NB,)),
        pltpu.SemaphoreType.DMA((NB,)),
    ],
    compiler_params=pltpu.CompilerParams(needs_layout_passes=False),
)
def _agg_kernel(y_hbm, src_hbm, dw_hbm, part_hbm,
                acc, iring, dwring, rows, sbuf,
                gsem, ssem, isem, dsem):
    c = lax.axis_index("c")
    s = lax.axis_index("s")
    wid = c * NS + s
    zero16 = jnp.zeros((16,), jnp.float32)

    def zz(i, _):
        for j in range(D // 16):
            rows[0, i, pl.ds(j * 16, 16)] = zero16
        return 0
    lax.fori_loop(0, ACHUNK, zz, 0)

    def zf(k, _):
        pltpu.async_copy(rows.at[0], acc.at[pl.ds(s * RPW + k * ACHUNK,
                                                  ACHUNK)], gsem.at[0])
        return 0
    lax.fori_loop(0, RPW // ACHUNK, zf, 0)

    def zw(k, _):
        pltpu.make_async_copy(rows.at[0], acc.at[pl.ds(s * RPW, ACHUNK)],
                              gsem.at[0]).wait()
        return 0
    lax.fori_loop(0, RPW // ACHUNK, zw, 0)

    for b in range(NB):
        pltpu.async_copy(src_hbm.at[wid, b], iring.at[b], isem.at[b])
        pltpu.async_copy(dw_hbm.at[wid, b], dwring.at[b], dsem.at[b])
    for b in range(NB):
        pltpu.make_async_copy(src_hbm.at[wid, b], iring.at[b],
                              isem.at[b]).wait()
        pltpu.async_copy(y_hbm.at[iring.at[b]], rows.at[b], gsem.at[b])
    plsc.subcore_barrier()

    def group(g, _):
        for k in range(UNROLL):
            it = g * UNROLL + k
            b = k % NB
            u = k % SB

            pltpu.make_async_copy(y_hbm.at[iring.at[b]], rows.at[b],
                                  gsem.at[b]).wait()

            def _fire_iload():
                pltpu.async_copy(src_hbm.at[wid, it + NB], iring.at[b],
                                 isem.at[b])
            if k < UNROLL - NB:
                _fire_iload()
            else:
                pl.when(g < NGRP - 1)(_fire_iload)

            bd = (k + 3) % NB

            def _drain_scatter():
                pltpu.make_async_copy(sbuf.at[u], acc.at[dwring.at[b, 0]],
                                      ssem.at[u]).wait()

            def _drain_and_dload():
                _drain_scatter()
                pltpu.async_copy(dw_hbm.at[wid, it + 3], dwring.at[bd],
                                 dsem.at[bd])
            if SB <= k < UNROLL - 3:
                _drain_and_dload()
            elif k < SB:
                pl.when(g > 0)(_drain_and_dload)
            else:
                pl.when(g < NGRP - 1)(_drain_and_dload)
                if k >= SB:
                    pl.when(g == NGRP - 1)(_drain_scatter)

            pltpu.make_async_copy(dw_hbm.at[wid, b], dwring.at[b],
                                  dsem.at[b]).wait()

            @plsc.parallel_loop(0, ACHUNK, step=1, unroll=4)
            def scale(i):
                ewv = plsc.bitcast(
                    plsc.load_gather(dwring.at[b, 1],
                                     [jnp.full((16,), i, jnp.int32)]),
                    jnp.float32)
                for j in range(D // 16):
                    sbuf[u, i, pl.ds(j * 16, 16)] = (
                        rows[b, i, pl.ds(j * 16, 16)] * ewv)

            pltpu.async_copy(sbuf.at[u], acc.at[dwring.at[b, 0]], ssem.at[u],
                             add=True)

            def _fire_gather():
                pltpu.make_async_copy(src_hbm.at[wid, b], iring.at[b],
                                      isem.at[b]).wait()
                pltpu.async_copy(y_hbm.at[iring.at[b]], rows.at[b],
                                 gsem.at[b])
            if k < UNROLL - NB:
                _fire_gather()
            else:
                pl.when(g < NGRP - 1)(_fire_gather)
        return 0
    lax.fori_loop(0, NGRP, group, 0)

    for u in range(SB):
        pltpu.make_async_copy(sbuf.at[u], acc.at[dwring.at[0, 0]],
                              ssem.at[u]).wait()
    plsc.subcore_barrier()
    pltpu.sync_copy(acc.at[pl.ds(s * RPW, RPW)],
                    part_hbm.at[c, pl.ds(s * RPW, RPW)])


def _dis_body(p0_ref, p1_ref, o_ref):
    o_ref[...] = lax.rsqrt(p0_ref[...] + p1_ref[...] + 1.0)


_dis_call = pl.pallas_call(
    _dis_body,
    out_shape=jax.ShapeDtypeStruct((NPAD // D, D), jnp.float32),
)

_RB = 1000


def _mm_scale_body(x_ref, w_ref, d_ref, o_ref):
    o_ref[...] = d_ref[...] * jnp.dot(
        x_ref[...], w_ref[...], preferred_element_type=jnp.float32)


_mm_scale_call = pl.pallas_call(
    _mm_scale_body,
    grid=(N // _RB,),
    in_specs=[
        pl.BlockSpec((_RB, D), lambda i: (i, 0)),
        pl.BlockSpec((D, D), lambda i: (0, 0)),
        pl.BlockSpec((_RB, 1), lambda i: (i, 0)),
    ],
    out_specs=pl.BlockSpec((_RB, D), lambda i: (i, 0)),
    out_shape=jax.ShapeDtypeStruct((N, D), jnp.float32),
)


def _comb_mm_body(p_ref, y_ref, d_ref, b_ref, w_ref, o_ref):
    h = (d_ref[...] * (p_ref[0] + p_ref[1] + y_ref[...]) + b_ref[...])
    h = jnp.maximum(h, 0.0)
    o_ref[...] = d_ref[...] * jnp.dot(
        h, w_ref[...], preferred_element_type=jnp.float32)


_comb_mm_call = pl.pallas_call(
    _comb_mm_body,
    grid=(N // _RB,),
    in_specs=[
        pl.BlockSpec((NC, _RB, D), lambda i: (0, i, 0)),
        pl.BlockSpec((_RB, D), lambda i: (i, 0)),
        pl.BlockSpec((_RB, 1), lambda i: (i, 0)),
        pl.BlockSpec((1, D), lambda i: (0, 0)),
        pl.BlockSpec((D, D), lambda i: (0, 0)),
    ],
    out_specs=pl.BlockSpec((_RB, D), lambda i: (i, 0)),
    out_shape=jax.ShapeDtypeStruct((N, D), jnp.float32),
)


def _comb_final_body(p_ref, y_ref, d_ref, b_ref, o_ref):
    o_ref[...] = (d_ref[...] * (p_ref[0] + p_ref[1] + y_ref[...])
                  + b_ref[...])


_comb_final_call = pl.pallas_call(
    _comb_final_body,
    grid=(N // _RB,),
    in_specs=[
        pl.BlockSpec((NC, _RB, D), lambda i: (0, i, 0)),
        pl.BlockSpec((_RB, D), lambda i: (i, 0)),
        pl.BlockSpec((_RB, 1), lambda i: (i, 0)),
        pl.BlockSpec((1, D), lambda i: (0, 0)),
    ],
    out_specs=pl.BlockSpec((_RB, D), lambda i: (i, 0)),
    out_shape=jax.ShapeDtypeStruct((N, D), jnp.float32),
)


def kernel(x, edge_index, edge_weight, W1, b1, W2, b2):
    src = edge_index[0]
    dst = edge_index[1]
    src3 = src.reshape(NW, ANITER, ACHUNK)
    dst3 = dst.reshape(NW, ANITER, ACHUNK)
    ewi3 = lax.bitcast_convert_type(edge_weight, jnp.int32).reshape(
        NW, ANITER, ACHUNK)
    dw = jnp.stack([dst3, ewi3], axis=2)

    pdeg = _deg_kernel(dst, edge_weight)
    dis2d = _dis_call(pdeg[0].reshape(NPAD // D, D), pdeg[1].reshape(NPAD // D, D))
    dis_col = dis2d.reshape(NPAD, 1)

    y1 = _mm_scale_call(x, W1, dis_col)
    part1 = _agg_kernel(y1, src3, dw)
    y2 = _comb_mm_call(part1, y1, dis_col, b1.reshape(1, D), W2)
    part2 = _agg_kernel(y2, src3, dw)
    out = _comb_final_call(part2, y2, dis_col, b2.reshape(1, D))
    return out


# speedup vs baseline: 1.0049x; 1.0049x over previous
import functools

import jax
import jax.numpy as jnp
from jax import lax
from jax.experimental import pallas as pl
from jax.experimental.pallas import tpu as pltpu
from jax.experimental.pallas import tpu_sc as plsc

N = 10000
E = 320000
D = 128
NC = 2
NS = 16
NW = NC * NS
EPW = E // NW
CHUNK = 80
NITER = EPW // CHUNK
NPAD = 10240
RPW = NPAD // NS
ZR = 32
DPW = NPAD // NS
ACHUNK = 40
ANITER = EPW // ACHUNK
NB = 5
SB = 2
UNROLL = 10
NGRP = ANITER // UNROLL

_mesh = plsc.VectorSubcoreMesh(core_axis_name="c", subcore_axis_name="s")


@functools.partial(
    pl.kernel,
    out_type=jax.ShapeDtypeStruct((NC, NPAD), jnp.float32),
    mesh=_mesh,
    scratch_types=[
        pltpu.VMEM((EPW,), jnp.int32),
        pltpu.VMEM((EPW,), jnp.float32),
        pltpu.VMEM((NPAD,), jnp.float32),
        pltpu.VMEM_SHARED((NS, NPAD), jnp.float32),
        pltpu.VMEM((DPW,), jnp.float32),
        pltpu.VMEM((DPW,), jnp.float32),
        pltpu.SemaphoreType.DMA,
        pltpu.SemaphoreType.DMA,
    ],
    compiler_params=pltpu.CompilerParams(needs_layout_passes=False),
)
def _deg_kernel(dst_hbm, ew_hbm, pdeg_hbm, dstb, ewb, ldeg, share, rowb, accb,
                dsem, esem):
    c = lax.axis_index("c")
    s = lax.axis_index("s")
    wid = c * NS + s
    zero16 = jnp.zeros((16,), jnp.float32)

    def zb(i, _):
        ldeg[pl.ds(i * 16, 16)] = zero16
        return 0
    lax.fori_loop(0, NPAD // 16, zb, 0)

    ebase = wid * EPW
    pltpu.async_copy(dst_hbm.at[pl.ds(ebase, EPW)], dstb, dsem)
    pltpu.async_copy(ew_hbm.at[pl.ds(ebase, EPW)], ewb, esem)
    pltpu.make_async_copy(dst_hbm.at[pl.ds(ebase, EPW)], dstb, dsem).wait()
    pltpu.make_async_copy(ew_hbm.at[pl.ds(ebase, EPW)], ewb, esem).wait()

    @plsc.parallel_loop(0, EPW // 16, step=1, unroll=4)
    def body(i):
        d16 = dstb[pl.ds(i * 16, 16)]
        w16 = ewb[pl.ds(i * 16, 16)]
        plsc.addupdate_scatter(ldeg, [d16], w16)

    pltpu.sync_copy(ldeg, share.at[s])
    plsc.subcore_barrier()
    base = s * DPW

    def za(k, _):
        accb[pl.ds(k * 16, 16)] = zero16
        return 0
    lax.fori_loop(0, DPW // 16, za, 0)

    def rsum(r, _):
        pltpu.sync_copy(share.at[r, pl.ds(base, DPW)], rowb)

        def vadd(k, _):
            accb[pl.ds(k * 16, 16)] = accb[pl.ds(k * 16, 16)] + rowb[pl.ds(k * 16, 16)]
            return 0
        lax.fori_loop(0, DPW // 16, vadd, 0)
        return 0
    lax.fori_loop(0, NS, rsum, 0)

    pltpu.sync_copy(accb, pdeg_hbm.at[c, pl.ds(base, DPW)])


@functools.partial(
    pl.kernel,
    out_type=jax.ShapeDtypeStruct((NC, NPAD, D), jnp.float32),
    mesh=_mesh,
    scratch_types=[
        pltpu.VMEM_SHARED((NPAD, D), jnp.float32),
        pltpu.VMEM((NB, ACHUNK), jnp.int32),
        pltpu.VMEM((NB, 2, ACHUNK), jnp.int32),
        pltpu.VMEM((NB, ACHUNK, D), jnp.float32),
        pltpu.VMEM((SB, ACHUNK, D), jnp.float32),
        pltpu.SemaphoreType.DMA((NB,)),
        pltpu.SemaphoreType.DMA((SB,)),
        pltpu.SemaphoreType.DMA((NB,)),
        pltpu.SemaphoreType.DMA((NB,)),
    ],
    compiler_params=pltpu.CompilerParams(needs_layout_passes=False),
)
def _agg_kernel(y_hbm, src_hbm, dw_hbm, part_hbm,
                acc, iring, dwring, rows, sbuf,
                gsem, ssem, isem, dsem):
    c = lax.axis_index("c")
    s = lax.axis_index("s")
    wid = c * NS + s
    zero16 = jnp.zeros((16,), jnp.float32)

    def zz(i, _):
        for j in range(D // 16):
            rows[0, i, pl.ds(j * 16, 16)] = zero16
        return 0
    lax.fori_loop(0, ACHUNK, zz, 0)

    def zf(k, _):
        pltpu.async_copy(rows.at[0], acc.at[pl.ds(s * RPW + k * ACHUNK,
                                                  ACHUNK)], gsem.at[0])
        return 0
    lax.fori_loop(0, RPW // ACHUNK, zf, 0)

    def zw(k, _):
        pltpu.make_async_copy(rows.at[0], acc.at[pl.ds(s * RPW, ACHUNK)],
                              gsem.at[0]).wait()
        return 0
    lax.fori_loop(0, RPW // ACHUNK, zw, 0)

    for b in range(NB):
        pltpu.async_copy(src_hbm.at[wid, b], iring.at[b], isem.at[b])
        pltpu.async_copy(dw_hbm.at[wid, b], dwring.at[b], dsem.at[b])
    for b in range(NB):
        pltpu.make_async_copy(src_hbm.at[wid, b], iring.at[b],
                              isem.at[b]).wait()
        pltpu.async_copy(y_hbm.at[iring.at[b]], rows.at[b], gsem.at[b])
    plsc.subcore_barrier()

    def group(g, _):
        for k in range(UNROLL):
            it = g * UNROLL + k
            b = k % NB
            u = k % SB

            pltpu.make_async_copy(y_hbm.at[iring.at[b]], rows.at[b],
                                  gsem.at[b]).wait()

            def _fire_iload():
                pltpu.async_copy(src_hbm.at[wid, it + NB], iring.at[b],
                                 isem.at[b])
            if k < UNROLL - NB:
                _fire_iload()
            else:
                pl.when(g < NGRP - 1)(_fire_iload)

            bd = (k + 3) % NB

            def _drain_scatter():
                pltpu.make_async_copy(sbuf.at[u], acc.at[dwring.at[b, 0]],
                                      ssem.at[u]).wait()

            def _drain_and_dload():
                _drain_scatter()
                pltpu.async_copy(dw_hbm.at[wid, it + 3], dwring.at[bd],
                                 dsem.at[bd])
            if SB <= k < UNROLL - 3:
                _drain_and_dload()
            elif k < SB:
                pl.when(g > 0)(_drain_and_dload)
            else:
                pl.when(g < NGRP - 1)(_drain_and_dload)
                if k >= SB:
                    pl.when(g == NGRP - 1)(_drain_scatter)

            pltpu.make_async_copy(dw_hbm.at[wid, b], dwring.at[b],
                                  dsem.at[b]).wait()

            @plsc.parallel_loop(0, ACHUNK, step=1, unroll=5)
            def scale(i):
                ewv = plsc.bitcast(
                    plsc.load_gather(dwring.at[b, 1],
                                     [jnp.full((16,), i, jnp.int32)]),
                    jnp.float32)
                for j in range(D // 16):
                    sbuf[u, i, pl.ds(j * 16, 16)] = (
                        rows[b, i, pl.ds(j * 16, 16)] * ewv)

            pltpu.async_copy(sbuf.at[u], acc.at[dwring.at[b, 0]], ssem.at[u],
                             add=True)

            def _fire_gather():
                pltpu.make_async_copy(src_hbm.at[wid, b], iring.at[b],
                                      isem.at[b]).wait()
                pltpu.async_copy(y_hbm.at[iring.at[b]], rows.at[b],
                                 gsem.at[b])
            if k < UNROLL - NB:
                _fire_gather()
            else:
                pl.when(g < NGRP - 1)(_fire_gather)
        return 0
    lax.fori_loop(0, NGRP, group, 0)

    for u in range(SB):
        pltpu.make_async_copy(sbuf.at[u], acc.at[dwring.at[0, 0]],
                              ssem.at[u]).wait()
    plsc.subcore_barrier()
    pltpu.sync_copy(acc.at[pl.ds(s * RPW, RPW)],
                    part_hbm.at[c, pl.ds(s * RPW, RPW)])


def _dis_body(p0_ref, p1_ref, o_ref):
    o_ref[...] = lax.rsqrt(p0_ref[...] + p1_ref[...] + 1.0)


_dis_call = pl.pallas_call(
    _dis_body,
    out_shape=jax.ShapeDtypeStruct((NPAD // D, D), jnp.float32),
)

_RB = 1000


def _mm_scale_body(x_ref, w_ref, d_ref, o_ref):
    o_ref[...] = d_ref[...] * jnp.dot(
        x_ref[...], w_ref[...], preferred_element_type=jnp.float32)


_mm_scale_call = pl.pallas_call(
    _mm_scale_body,
    grid=(N // _RB,),
    in_specs=[
        pl.BlockSpec((_RB, D), lambda i: (i, 0)),
        pl.BlockSpec((D, D), lambda i: (0, 0)),
        pl.BlockSpec((_RB, 1), lambda i: (i, 0)),
    ],
    out_specs=pl.BlockSpec((_RB, D), lambda i: (i, 0)),
    out_shape=jax.ShapeDtypeStruct((N, D), jnp.float32),
)


def _comb_mm_body(p_ref, y_ref, d_ref, b_ref, w_ref, o_ref):
    h = (d_ref[...] * (p_ref[0] + p_ref[1] + y_ref[...]) + b_ref[...])
    h = jnp.maximum(h, 0.0)
    o_ref[...] = d_ref[...] * jnp.dot(
        h, w_ref[...], preferred_element_type=jnp.float32)


_comb_mm_call = pl.pallas_call(
    _comb_mm_body,
    grid=(N // _RB,),
    in_specs=[
        pl.BlockSpec((NC, _RB, D), lambda i: (0, i, 0)),
        pl.BlockSpec((_RB, D), lambda i: (i, 0)),
        pl.BlockSpec((_RB, 1), lambda i: (i, 0)),
        pl.BlockSpec((1, D), lambda i: (0, 0)),
        pl.BlockSpec((D, D), lambda i: (0, 0)),
    ],
    out_specs=pl.BlockSpec((_RB, D), lambda i: (i, 0)),
    out_shape=jax.ShapeDtypeStruct((N, D), jnp.float32),
)


def _comb_final_body(p_ref, y_ref, d_ref, b_ref, o_ref):
    o_ref[...] = (d_ref[...] * (p_ref[0] + p_ref[1] + y_ref[...])
                  + b_ref[...])


_comb_final_call = pl.pallas_call(
    _comb_final_body,
    grid=(N // _RB,),
    in_specs=[
        pl.BlockSpec((NC, _RB, D), lambda i: (0, i, 0)),
        pl.BlockSpec((_RB, D), lambda i: (i, 0)),
        pl.BlockSpec((_RB, 1), lambda i: (i, 0)),
        pl.BlockSpec((1, D), lambda i: (0, 0)),
    ],
    out_specs=pl.BlockSpec((_RB, D), lambda i: (i, 0)),
    out_shape=jax.ShapeDtypeStruct((N, D), jnp.float32),
)


def kernel(x, edge_index, edge_weight, W1, b1, W2, b2):
    src = edge_index[0]
    dst = edge_index[1]
    src3 = src.reshape(NW, ANITER, ACHUNK)
    dst3 = dst.reshape(NW, ANITER, ACHUNK)
    ewi3 = lax.bitcast_convert_type(edge_weight, jnp.int32).reshape(
        NW, ANITER, ACHUNK)
    dw = jnp.stack([dst3, ewi3], axis=2)

    pdeg = _deg_kernel(dst, edge_weight)
    dis2d = _dis_call(pdeg[0].reshape(NPAD // D, D), pdeg[1].reshape(NPAD // D, D))
    dis_col = dis2d.reshape(NPAD, 1)

    y1 = _mm_scale_call(x, W1, dis_col)
    part1 = _agg_kernel(y1, src3, dw)
    y2 = _comb_mm_call(part1, y1, dis_col, b1.reshape(1, D), W2)
    part2 = _agg_kernel(y2, src3, dw)
    out = _comb_final_call(part2, y2, dis_col, b2.reshape(1, D))
    return out
